# baseline (speedup 1.0000x reference)
import functools

import jax
import jax.numpy as jnp
from jax import lax
from jax.experimental import pallas as pl
from jax.experimental.pallas import tpu as pltpu
from jax.experimental.pallas import tpu_sc as plsc

NUM_CODES = 1296
CODE_DIM = 256
TILE_N = 512


def _vq_body(x_ref, emb_ref, x2_ref, codes_ref):
    x = x_ref[...]
    xm2 = -2.0 * x

    mm = lax.dot_general(xm2, emb_ref[...], (((1,), (1,)), ((), ())),
                         preferred_element_type=jnp.float32)
    x1 = jnp.sum(x * x, axis=1, keepdims=True)
    d = (x1 + x2_ref[...]) + mm

    m = jnp.min(d, axis=1, keepdims=True)
    kf = lax.broadcasted_iota(jnp.int32, (TILE_N, NUM_CODES), 1).astype(jnp.float32)
    idxf = jnp.min(jnp.where(d == m, kf, jnp.float32(3e38)), axis=1)
    codes_ref[0, 0, :] = idxf.astype(jnp.int32)


def _argmin_codes(xf, emb, x2row):
    total = xf.shape[0]
    nb = total // TILE_N
    codes3d = pl.pallas_call(
        _vq_body,
        grid=(nb,),
        in_specs=[
            pl.BlockSpec((TILE_N, CODE_DIM), lambda i: (i, 0)),
            pl.BlockSpec((NUM_CODES, CODE_DIM), lambda i: (0, 0)),
            pl.BlockSpec((1, NUM_CODES), lambda i: (0, 0)),
        ],
        out_specs=pl.BlockSpec((1, 1, TILE_N), lambda i: (i, 0, 0)),
        out_shape=jax.ShapeDtypeStruct((nb, 1, TILE_N), jnp.int32),
    )(xf, emb, x2row)
    return codes3d.reshape(total)


def _sc_gather(table, idx_flat):
    info = plsc.get_sparse_core_info()
    nc, ns = info.num_cores, info.num_subcores
    nw = nc * ns
    total = idx_flat.shape[0]
    b_per_w = total // nw
    chunk = min(128, b_per_w)
    n_chunks = b_per_w // chunk
    mesh = plsc.VectorSubcoreMesh(core_axis_name="c", subcore_axis_name="s")

    @functools.partial(
        pl.kernel, mesh=mesh,
        out_type=jax.ShapeDtypeStruct((total, CODE_DIM), jnp.float32),
        scratch_types=[
            pltpu.VMEM((chunk,), jnp.int32),
            pltpu.VMEM((chunk,), jnp.int32),
            pltpu.VMEM((chunk, CODE_DIM), jnp.float32),
            pltpu.VMEM((chunk, CODE_DIM), jnp.float32),
            pltpu.SemaphoreType.DMA,
            pltpu.SemaphoreType.DMA,
            pltpu.SemaphoreType.DMA,
            pltpu.SemaphoreType.DMA,
        ],
    )
    def gather_k(table_hbm, idx_hbm, out_hbm,
                 idx0, idx1, rows0, rows1, gs0, gs1, ws0, ws1):
        wid = lax.axis_index("s") * nc + lax.axis_index("c")
        base = wid * b_per_w
        bufs = ((idx0, rows0, gs0, ws0), (idx1, rows1, gs1, ws1))
        writeback = [None, None]
        for c in range(n_chunks):
            bi = c & 1
            idx_v, rows_v, gsem, wsem = bufs[bi]
            if writeback[bi] is not None:
                writeback[bi].wait()
            off = base + c * chunk
            pltpu.sync_copy(idx_hbm.at[pl.ds(off, chunk)], idx_v)
            pltpu.async_copy(table_hbm.at[idx_v], rows_v, gsem).wait()
            writeback[bi] = pltpu.async_copy(
                rows_v, out_hbm.at[pl.ds(off, chunk)], wsem)
        for wb in writeback:
            if wb is not None:
                wb.wait()

    return gather_k(table, idx_flat)


@jax.jit
def kernel(inputs, embeddings):
    b, m, n, d = inputs.shape
    total = b * m * n
    xf = inputs.reshape(total, d)
    x2row = jnp.sum(embeddings ** 2, axis=-1)[None, :]

    codes_flat = _argmin_codes(xf, embeddings, x2row)
    vecs = _sc_gather(embeddings, codes_flat)

    return (codes_flat.reshape(b, m, n), vecs.reshape(b, m, n, d))
